# Initial kernel scaffold; baseline (speedup 1.0000x reference)
import jax
import jax.numpy as jnp
from jax.experimental import pallas as pl


def kernel(result, input, lora_a, lora_b, adapter_indices):
    raise NotImplementedError("write your pallas kernel here")


# speedup vs baseline: 6.6977x; 6.6977x over previous
import jax
import jax.numpy as jnp
from jax.experimental import pallas as pl


def _body(x_ref, res_ref, a_ref, b_ref, idx_ref, out_ref, *, E, R):
    B = x_ref.shape[0]
    ER = E * R
    x = x_ref[...].astype(jnp.bfloat16)
    v = jax.lax.dot_general(
        x, a_ref[...],
        dimension_numbers=(((1,), (1,)), ((), ())),
        preferred_element_type=jnp.float32,
    )
    idx = idx_ref[0]
    lane_adapter = jax.lax.broadcasted_iota(jnp.int32, (B, ER), 1) // R
    vm = jnp.where(lane_adapter == idx, v, 0.0).astype(jnp.bfloat16)
    y = jax.lax.dot_general(
        vm, b_ref[...],
        dimension_numbers=(((1,), (0,)), ((), ())),
        preferred_element_type=jnp.float32,
    )
    out_ref[...] = res_ref[...] + y


def kernel(result, input, lora_a, lora_b, adapter_indices):
    N, D = input.shape
    E, _L, R, _D = lora_a.shape
    ER = E * R
    B = 512 if N % 512 == 0 else 256
    NB = N // B

    a_bf = lora_a[:, 0].reshape(ER, D).astype(jnp.bfloat16)
    b_bf = lora_b[:, 0].reshape(ER, D).astype(jnp.bfloat16)
    idx3 = adapter_indices.astype(jnp.int32).reshape(NB, B, 1)

    import functools
    body = functools.partial(_body, E=E, R=R)

    out = pl.pallas_call(
        body,
        grid=(NB,),
        in_specs=[
            pl.BlockSpec((B, D), lambda i: (i, 0)),
            pl.BlockSpec((B, D), lambda i: (i, 0)),
            pl.BlockSpec((ER, D), lambda i: (0, 0)),
            pl.BlockSpec((ER, D), lambda i: (0, 0)),
            pl.BlockSpec((1, B, 1), lambda i: (i, 0, 0)),
        ],
        out_specs=pl.BlockSpec((B, D), lambda i: (i, 0)),
        out_shape=jax.ShapeDtypeStruct((N, D), jnp.float32),
    )(input, result, a_bf, b_bf, idx3)
    return out
